# baseline (speedup 1.0000x reference)
import jax
import jax.numpy as jnp
from jax.experimental import pallas as pl
from jax.experimental.pallas import tpu as pltpu

_T = 256


def _body(x_ref, id_ref, out_ref, L_ref, c_ref, r_ref, sid_ref, *, nt):
    p = pl.program_id(0)
    i = pl.program_id(1)
    T = x_ref.shape[0]
    neg = jnp.float32(-jnp.inf)

    ids = id_ref[...]
    first_id = id_ref[0, 0]
    last_id = id_ref[T - 1, 0]
    single = first_id == last_id
    x = x_ref[...]

    @pl.when(p == 0)
    def _phase0():
        c_id_prev = jnp.where(i == 0, -1, sid_ref[0])
        c_vec_prev = c_ref[0:1, :]
        l_row = jnp.where(first_id == c_id_prev, c_vec_prev,
                          jnp.full_like(c_vec_prev, neg))
        L_ref[pl.ds(i, 1), :] = l_row
        tailmax = jnp.max(jnp.where(ids == last_id, x, neg), axis=0,
                          keepdims=True)
        c_ref[0:1, :] = jnp.maximum(
            tailmax, jnp.where(single, l_row, jnp.full_like(l_row, neg)))
        sid_ref[0] = last_id

    @pl.when(p == 1)
    def _phase1():
        j = nt - 1 - i
        r_id_prev = jnp.where(i == 0, -1, sid_ref[1])
        r_vec_prev = r_ref[0:1, :]

        f = x
        b = x
        k = 1
        while k < T:
            pad_i = jnp.full((k, 1), -1, jnp.int32)
            pad_f = jnp.full((k, x.shape[1]), neg, jnp.float32)
            same_d = ids == jnp.concatenate([pad_i, ids[:T - k]], axis=0)
            f_sh = jnp.concatenate([pad_f, f[:T - k]], axis=0)
            f = jnp.where(same_d, jnp.maximum(f, f_sh), f)
            same_u = ids == jnp.concatenate([ids[k:], pad_i], axis=0)
            b_sh = jnp.concatenate([b[k:], pad_f], axis=0)
            b = jnp.where(same_u, jnp.maximum(b, b_sh), b)
            k *= 2

        m = jnp.maximum(f, b)
        l_row = L_ref[pl.ds(j, 1), :]
        m = jnp.where(ids == first_id, jnp.maximum(m, l_row), m)
        m = jnp.where(ids == r_id_prev, jnp.maximum(m, r_vec_prev), m)
        out_ref[...] = m

        r_new = jnp.maximum(
            b[0:1, :],
            jnp.where(jnp.logical_and(single, first_id == r_id_prev),
                      r_vec_prev, jnp.full_like(r_vec_prev, neg)))
        r_ref[0:1, :] = r_new
        sid_ref[1] = first_id


def kernel(x_sp, vertex_id):
    E, D = x_sp.shape
    T = _T
    nt = E // T
    idcol = vertex_id.reshape(E, 1)

    import functools
    body = functools.partial(_body, nt=nt)

    def x_map(p, i):
        return (jnp.where(p == 0, i, nt - 1 - i), 0)

    def out_map(p, i):
        return (jnp.where(p == 0, nt - 1, nt - 1 - i), 0)

    return pl.pallas_call(
        body,
        grid=(2, nt),
        in_specs=[
            pl.BlockSpec((T, D), x_map),
            pl.BlockSpec((T, 1), x_map),
        ],
        out_specs=pl.BlockSpec((T, D), out_map),
        out_shape=jax.ShapeDtypeStruct((E, D), jnp.float32),
        scratch_shapes=[
            pltpu.VMEM((nt, D), jnp.float32),
            pltpu.VMEM((1, D), jnp.float32),
            pltpu.VMEM((1, D), jnp.float32),
            pltpu.SMEM((2,), jnp.int32),
        ],
    )(x_sp, idcol)


# speedup vs baseline: 1.0738x; 1.0738x over previous
import functools

import jax
import jax.numpy as jnp
from jax.experimental import pallas as pl
from jax.experimental.pallas import tpu as pltpu

_T = 256


def _roll_down(a, k):
    return jnp.concatenate([a[a.shape[0] - k:], a[:a.shape[0] - k]], axis=0)


def _roll_up(a, k):
    return jnp.concatenate([a[k:], a[:k]], axis=0)


def _body(x_ref, id_ref, out_ref, L_ref, c_ref, r_ref, sid_ref, *, nt):
    p = pl.program_id(0)
    i = pl.program_id(1)
    T = x_ref.shape[0]
    D = x_ref.shape[1]
    neg = jnp.float32(-jnp.inf)

    idcol = id_ref[...]
    idb = jnp.broadcast_to(idcol, (T, D))
    first_id = id_ref[0, 0]
    last_id = id_ref[T - 1, 0]
    single = first_id == last_id
    x = x_ref[...]

    @pl.when(p == 0)
    def _phase0():
        c_id_prev = jnp.where(i == 0, -1, sid_ref[0])
        c_vec_prev = c_ref[0:1, :]
        l_row = jnp.where(first_id == c_id_prev, c_vec_prev,
                          jnp.full_like(c_vec_prev, neg))
        L_ref[pl.ds(i, 1), :] = l_row
        tailmax = jnp.max(jnp.where(idb == last_id, x, neg), axis=0,
                          keepdims=True)
        c_ref[0:1, :] = jnp.maximum(
            tailmax, jnp.where(single, l_row, jnp.full_like(l_row, neg)))
        sid_ref[0] = last_id

    @pl.when(p == 1)
    def _phase1():
        j = nt - 1 - i
        r_id_prev = jnp.where(i == 0, -1, sid_ref[1])
        r_vec_prev = r_ref[0:1, :]

        f = x
        b = x
        k = 1
        while k < T:
            same_d = idb == _roll_down(idb, k)
            f = jnp.where(same_d, jnp.maximum(f, _roll_down(f, k)), f)
            same_u = idb == _roll_up(idb, k)
            b = jnp.where(same_u, jnp.maximum(b, _roll_up(b, k)), b)
            k *= 2

        m = jnp.maximum(f, b)
        l_row = L_ref[pl.ds(j, 1), :]
        m = jnp.where(idb == first_id, jnp.maximum(m, l_row), m)
        m = jnp.where(idb == r_id_prev, jnp.maximum(m, r_vec_prev), m)
        out_ref[...] = m

        r_new = jnp.maximum(
            b[0:1, :],
            jnp.where(jnp.logical_and(single, first_id == r_id_prev),
                      r_vec_prev, jnp.full_like(r_vec_prev, neg)))
        r_ref[0:1, :] = r_new
        sid_ref[1] = first_id


def kernel(x_sp, vertex_id):
    E, D = x_sp.shape
    T = _T
    nt = E // T
    idcol = vertex_id.reshape(E, 1)

    body = functools.partial(_body, nt=nt)

    def x_map(p, i):
        return (jnp.where(p == 0, i, nt - 1 - i), 0)

    def out_map(p, i):
        return (jnp.where(p == 0, nt - 1, nt - 1 - i), 0)

    return pl.pallas_call(
        body,
        grid=(2, nt),
        in_specs=[
            pl.BlockSpec((T, D), x_map),
            pl.BlockSpec((T, 1), x_map),
        ],
        out_specs=pl.BlockSpec((T, D), out_map),
        out_shape=jax.ShapeDtypeStruct((E, D), jnp.float32),
        scratch_shapes=[
            pltpu.VMEM((nt, D), jnp.float32),
            pltpu.VMEM((1, D), jnp.float32),
            pltpu.VMEM((1, D), jnp.float32),
            pltpu.SMEM((2,), jnp.int32),
        ],
    )(x_sp, idcol)
